# Initial kernel scaffold; baseline (speedup 1.0000x reference)
import jax
import jax.numpy as jnp
from jax import lax
from jax.experimental import pallas as pl
from jax.experimental.pallas import tpu as pltpu
from jax.experimental.pallas import tpu_sc as plsc

N = 10000
D = 128
E = 320000
NC = 2
NS = 16
NW = NC * NS
EPW = E // NW
CH = 125
NCH = EPW // CH
RPT = N // NS

ROWS_BLK = 1000
GRID = N // ROWS_BLK


def _agg_body(x_hbm, srcs, dsts, zeros_hbm, parts,
              src_v, dst_v, rows_a, rows_b, sem_a, sem_b, agg_sh):
    cid = lax.axis_index("c")
    sid = lax.axis_index("s")
    wid = cid * NS + sid

    pltpu.sync_copy(srcs.at[wid], src_v)
    pltpu.sync_copy(dsts.at[wid], dst_v)

    row0 = sid * RPT
    use_x = (sid < NS // 2) == (cid == 0)

    @pl.when(use_x)
    def _():
        pltpu.sync_copy(x_hbm.at[pl.ds(row0, RPT)], agg_sh.at[pl.ds(row0, RPT)])

    @pl.when(jnp.logical_not(use_x))
    def _():
        pltpu.sync_copy(zeros_hbm, agg_sh.at[pl.ds(row0, RPT)])

    plsc.subcore_barrier()

    def gather_start(j, buf, sem):
        pltpu.async_copy(x_hbm.at[src_v.at[j]], buf, sem)

    def gather_wait(j, buf, sem):
        pltpu.make_async_copy(x_hbm.at[src_v.at[j]], buf, sem).wait()

    def scatter_add(j, buf):
        pltpu.sync_copy(buf, agg_sh.at[dst_v.at[j]], add=True)

    gather_start(0, rows_a, sem_a)
    gather_start(1, rows_b, sem_b)

    def body(i, carry):
        g = 2 * i
        gather_wait(g, rows_a, sem_a)
        scatter_add(g, rows_a)

        @pl.when(g + 2 < NCH)
        def _():
            gather_start(g + 2, rows_a, sem_a)

        gather_wait(g + 1, rows_b, sem_b)
        scatter_add(g + 1, rows_b)

        @pl.when(g + 3 < NCH)
        def _():
            gather_start(g + 3, rows_b, sem_b)

        return carry

    lax.fori_loop(0, NCH // 2, body, 0)

    plsc.subcore_barrier()
    pltpu.sync_copy(agg_sh.at[pl.ds(row0, RPT)],
                    parts.at[cid, pl.ds(row0, RPT)])


def _mlp_body(pa_ref, pb_ref, w1_ref, b1_ref, w2_ref, b2_ref, o_ref):
    t = pa_ref[0] + pb_ref[0]
    h = jnp.dot(t, w1_ref[...], preferred_element_type=jnp.float32)
    h = jnp.maximum(h + b1_ref[...], 0.0)
    o_ref[...] = (jnp.dot(h, w2_ref[...], preferred_element_type=jnp.float32)
                  + b2_ref[...])


def kernel(x, edge_index, W1, b1, W2, b2):
    src = edge_index[0].astype(jnp.int32).reshape(NW, NCH, CH)
    dst = edge_index[1].astype(jnp.int32).reshape(NW, NCH, CH)
    zeros = jnp.zeros((RPT, D), jnp.float32)

    agg = pl.kernel(
        _agg_body,
        out_type=jax.ShapeDtypeStruct((NC, N, D), jnp.float32),
        mesh=plsc.VectorSubcoreMesh(core_axis_name="c", subcore_axis_name="s"),
        scratch_types=[
            pltpu.VMEM((NCH, CH), jnp.int32),
            pltpu.VMEM((NCH, CH), jnp.int32),
            pltpu.VMEM((CH, D), jnp.float32),
            pltpu.VMEM((CH, D), jnp.float32),
            pltpu.SemaphoreType.DMA,
            pltpu.SemaphoreType.DMA,
            pltpu.VMEM_SHARED((N, D), jnp.float32),
        ],
    )
    parts = agg(x, src, dst, zeros)

    return pl.pallas_call(
        _mlp_body,
        grid=(GRID,),
        in_specs=[
            pl.BlockSpec((1, ROWS_BLK, D), lambda i: (0, i, 0)),
            pl.BlockSpec((1, ROWS_BLK, D), lambda i: (1, i, 0)),
            pl.BlockSpec((D, D), lambda i: (0, 0)),
            pl.BlockSpec((1, D), lambda i: (0, 0)),
            pl.BlockSpec((D, D), lambda i: (0, 0)),
            pl.BlockSpec((1, D), lambda i: (0, 0)),
        ],
        out_specs=pl.BlockSpec((ROWS_BLK, D), lambda i: (i, 0)),
        out_shape=jax.ShapeDtypeStruct((N, D), jnp.float32),
    )(parts, parts, W1, b1.reshape(1, D), W2, b2.reshape(1, D))


# speedup vs baseline: 7.4394x; 7.4394x over previous
import jax
import jax.numpy as jnp
from jax import lax
from jax.experimental import pallas as pl
from jax.experimental.pallas import tpu as pltpu
from jax.experimental.pallas import tpu_sc as plsc

N = 10000
D = 128
E = 320000
NC = 2
NS = 16
NW = NC * NS
CH = 128
NCH = 80
EPW = NCH * CH
E_PAD = NW * EPW

NA = 9712
NA_ACC = NA + 8
CPT = 600
NB = N - NA
NB_ACC = NB + 8

ROWS_BLK = 1000
GRID = N // ROWS_BLK


def _zero_buf(buf, rows):
    def zrow(i, carry):
        for c in range(D // 16):
            buf[i, pl.ds(c * 16, 16)] = jnp.zeros((16,), jnp.float32)
        return carry

    lax.fori_loop(0, rows, zrow, 0)


def _agg_a_body(x_hbm, srcs, dsts, parts,
                src_v, dst_v, rows_a, rows_b, sem_a, sem_b, agg_sh):
    cid = lax.axis_index("c")
    sid = lax.axis_index("s")
    wid = cid * NS + sid

    pltpu.sync_copy(srcs.at[wid], src_v)
    pltpu.sync_copy(dsts.at[wid], dst_v)

    def remap(r, carry):
        for c in range(CH // 16):
            v = dst_v[r, pl.ds(c * 16, 16)]
            dmp = NA + (v & 7)
            dst_v[r, pl.ds(c * 16, 16)] = jnp.where(v < NA, v, dmp)
        return carry

    lax.fori_loop(0, NCH, remap, 0)

    _zero_buf(rows_a, CH)
    z0 = sid * 607
    for p in range(5):
        cnt = 128 if p < 4 else 607 - 4 * 128
        pltpu.sync_copy(rows_a.at[pl.ds(0, cnt)],
                        agg_sh.at[pl.ds(z0 + p * 128, cnt)])

    @pl.when(sid == NS - 1)
    def _():
        t0 = NS * 607
        pltpu.sync_copy(rows_a.at[pl.ds(0, NA_ACC - t0)],
                        agg_sh.at[pl.ds(t0, NA_ACC - t0)])

    plsc.subcore_barrier()

    def gather_start(j, buf, sem):
        pltpu.async_copy(x_hbm.at[src_v.at[j]], buf, sem)

    def gather_wait(j, buf, sem):
        pltpu.make_async_copy(x_hbm.at[src_v.at[j]], buf, sem).wait()

    def scatter_add(j, buf):
        pltpu.sync_copy(buf, agg_sh.at[dst_v.at[j]], add=True)

    gather_start(0, rows_a, sem_a)
    gather_start(1, rows_b, sem_b)

    def body(i, carry):
        g = 2 * i
        gather_wait(g, rows_a, sem_a)
        scatter_add(g, rows_a)

        @pl.when(g + 2 < NCH)
        def _():
            gather_start(g + 2, rows_a, sem_a)

        gather_wait(g + 1, rows_b, sem_b)
        scatter_add(g + 1, rows_b)

        @pl.when(g + 3 < NCH)
        def _():
            gather_start(g + 3, rows_b, sem_b)

        return carry

    lax.fori_loop(0, NCH // 2, body, 0)

    plsc.subcore_barrier()

    out0 = pl.multiple_of(sid * CPT, 8)
    pltpu.sync_copy(agg_sh.at[pl.ds(out0, CPT)],
                    parts.at[cid, pl.ds(out0, CPT)])

    @pl.when(sid == NS - 1)
    def _():
        tail = NS * CPT
        pltpu.sync_copy(agg_sh.at[pl.ds(tail, NA - tail)],
                        parts.at[cid, pl.ds(tail, NA - tail)])


def _agg_b_body(x_hbm, srcs, dsts, parts,
                src_v, dst_v, rows_a, rows_b, sem_a, sem_b, agg_sh):
    cid = lax.axis_index("c")
    sid = lax.axis_index("s")
    wid = cid * NS + sid

    pltpu.sync_copy(srcs.at[wid], src_v)
    pltpu.sync_copy(dsts.at[wid], dst_v)

    def remap(r, carry):
        for c in range(CH // 16):
            v = dst_v[r, pl.ds(c * 16, 16)]
            m = jnp.logical_and(v >= NA, v < N)
            dmp = NB + (v & 7)
            dst_v[r, pl.ds(c * 16, 16)] = jnp.where(m, v - NA, dmp)
        return carry

    lax.fori_loop(0, NCH, remap, 0)

    _zero_buf(rows_a, CH)

    @pl.when(sid == 0)
    def _():
        for p, cnt_p in ((0, 128), (1, 128), (2, NB_ACC - 256)):
            pltpu.sync_copy(rows_a.at[pl.ds(0, cnt_p)],
                            agg_sh.at[pl.ds(p * 128, cnt_p)])

    plsc.subcore_barrier()

    def gather_start(j, buf, sem):
        pltpu.async_copy(x_hbm.at[src_v.at[j]], buf, sem)

    def gather_wait(j, buf, sem):
        pltpu.make_async_copy(x_hbm.at[src_v.at[j]], buf, sem).wait()

    def scatter_add(j, buf):
        pltpu.sync_copy(buf, agg_sh.at[dst_v.at[j]], add=True)

    gather_start(0, rows_a, sem_a)
    gather_start(1, rows_b, sem_b)

    def body(i, carry):
        g = 2 * i
        gather_wait(g, rows_a, sem_a)
        scatter_add(g, rows_a)

        @pl.when(g + 2 < NCH)
        def _():
            gather_start(g + 2, rows_a, sem_a)

        gather_wait(g + 1, rows_b, sem_b)
        scatter_add(g + 1, rows_b)

        @pl.when(g + 3 < NCH)
        def _():
            gather_start(g + 3, rows_b, sem_b)

        return carry

    lax.fori_loop(0, NCH // 2, body, 0)

    plsc.subcore_barrier()

    @pl.when(sid == 0)
    def _():
        pltpu.sync_copy(agg_sh.at[pl.ds(0, NB)], parts.at[cid])


def _mlp_body(pa_ref, pb_ref, x_ref, w1_ref, b1_ref, w2_ref, b2_ref, o_ref):
    t = pa_ref[0] + pb_ref[0] + x_ref[...]
    h = jnp.dot(t, w1_ref[...], preferred_element_type=jnp.float32)
    h = jnp.maximum(h + b1_ref[...], 0.0)
    o_ref[...] = (jnp.dot(h, w2_ref[...], preferred_element_type=jnp.float32)
                  + b2_ref[...])


def kernel(x, edge_index, W1, b1, W2, b2):
    src = edge_index[0].astype(jnp.int32)
    dst = edge_index[1].astype(jnp.int32)
    npad = E_PAD - E
    pad_pos = jnp.arange(npad, dtype=jnp.int32)
    src_p = jnp.concatenate([src, pad_pos & 4095]).reshape(NW, NCH, CH)
    dst_p = jnp.concatenate([dst, jnp.full((npad,), N, jnp.int32)]
                            ).reshape(NW, NCH, CH)

    mesh = plsc.VectorSubcoreMesh(core_axis_name="c", subcore_axis_name="s")
    parts_a = pl.kernel(
        _agg_a_body,
        out_type=jax.ShapeDtypeStruct((NC, NA, D), jnp.float32),
        mesh=mesh,
        scratch_types=[
            pltpu.VMEM((NCH, CH), jnp.int32),
            pltpu.VMEM((NCH, CH), jnp.int32),
            pltpu.VMEM((CH, D), jnp.float32),
            pltpu.VMEM((CH, D), jnp.float32),
            pltpu.SemaphoreType.DMA,
            pltpu.SemaphoreType.DMA,
            pltpu.VMEM_SHARED((NA_ACC, D), jnp.float32),
        ],
    )(x, src_p, dst_p)

    parts_b = pl.kernel(
        _agg_b_body,
        out_type=jax.ShapeDtypeStruct((NC, NB, D), jnp.float32),
        mesh=mesh,
        scratch_types=[
            pltpu.VMEM((NCH, CH), jnp.int32),
            pltpu.VMEM((NCH, CH), jnp.int32),
            pltpu.VMEM((CH, D), jnp.float32),
            pltpu.VMEM((CH, D), jnp.float32),
            pltpu.SemaphoreType.DMA,
            pltpu.SemaphoreType.DMA,
            pltpu.VMEM_SHARED((NB_ACC, D), jnp.float32),
        ],
    )(x, src_p, dst_p)

    parts = jnp.concatenate([parts_a, parts_b], axis=1)

    return pl.pallas_call(
        _mlp_body,
        grid=(GRID,),
        in_specs=[
            pl.BlockSpec((1, ROWS_BLK, D), lambda i: (0, i, 0)),
            pl.BlockSpec((1, ROWS_BLK, D), lambda i: (1, i, 0)),
            pl.BlockSpec((ROWS_BLK, D), lambda i: (i, 0)),
            pl.BlockSpec((D, D), lambda i: (0, 0)),
            pl.BlockSpec((1, D), lambda i: (0, 0)),
            pl.BlockSpec((D, D), lambda i: (0, 0)),
            pl.BlockSpec((1, D), lambda i: (0, 0)),
        ],
        out_specs=pl.BlockSpec((ROWS_BLK, D), lambda i: (i, 0)),
        out_shape=jax.ShapeDtypeStruct((N, D), jnp.float32),
    )(parts, parts, x, W1, b1.reshape(1, D), W2, b2.reshape(1, D))
